# Initial kernel scaffold; baseline (speedup 1.0000x reference)
import jax
import jax.numpy as jnp
from jax.experimental import pallas as pl


def kernel(feature, edge_index, label, W1, b1, W2, b2, W3, b3, Wp1, bp1, Wp2, bp2, Wm0, bm0, Wm1, bm1, Wm2, bm2, Wn0, bn0, Wn1, bn1, Wn2, bn2):
    raise NotImplementedError("write your pallas kernel here")


# speedup vs baseline: 5.3464x; 5.3464x over previous
import functools

import jax
import jax.numpy as jnp
import numpy as np
from jax import lax
from jax.experimental import pallas as pl
from jax.experimental.pallas import tpu as pltpu
from jax.experimental.pallas import tpu_sc as plsc

N = 10000
E = 320000
D = 128
DH = D // 2
NC = 2
NS = 16
NW = NC * NS
K = 80
NCH_T = E // NS // K
NCH_W = E // NW // K
NP = 10240
RPT = NP // NS
ZROWS = 128
W16 = 16
BLK = 2000
NBLK = N // BLK
f32 = jnp.float32

_KEEP_MASKED = float(np.float32(1.0) - np.float32(0.999))


def _zero_fill(zbuf, width):
    zero = jnp.zeros((16,), f32)

    def zrow(i, _):
        def zcol(j, _):
            zbuf[i, pl.ds(j * 16, 16)] = zero
            return 0
        return lax.fori_loop(0, width // 16, zcol, 0)

    lax.fori_loop(0, ZROWS, zrow, 0)


def _zero_shared(zbuf, acc, sid):
    def zcp(r, _):
        pltpu.sync_copy(zbuf, acc.at[pl.ds(sid * RPT + r * ZROWS, ZROWS)])
        return 0

    lax.fori_loop(0, RPT // ZROWS, zcp, 0)


@functools.lru_cache(maxsize=None)
def _make_agg_split():
    mesh = plsc.VectorSubcoreMesh(core_axis_name="c", subcore_axis_name="s")

    @functools.partial(
        pl.kernel,
        out_type=(jax.ShapeDtypeStruct((NP, DH), f32),
                  jax.ShapeDtypeStruct((NP, DH), f32)),
        mesh=mesh,
        compiler_params=pltpu.CompilerParams(use_tc_tiling_on_sc=False),
        scratch_types=[
            pltpu.VMEM((NCH_T, K), jnp.int32),
            pltpu.VMEM((NCH_T, K), jnp.int32),
            pltpu.VMEM((K, DH), f32),
            pltpu.VMEM((ZROWS, DH), f32),
            pltpu.VMEM_SHARED((NP, DH), f32),
            pltpu.SemaphoreType.DMA,
        ],
    )
    def agg(hA, hB, src_hbm, dst_hbm, outA, outB,
            src_v, dst_v, gbuf, zbuf, acc, sem):
        cid = lax.axis_index("c")
        sid = lax.axis_index("s")

        pltpu.sync_copy(src_hbm.at[sid], src_v)
        pltpu.sync_copy(dst_hbm.at[sid], dst_v)

        _zero_fill(zbuf, DH)
        _zero_shared(zbuf, acc, sid)
        plsc.subcore_barrier()

        def chunk(c, _):
            @pl.when(cid == 0)
            def _():
                pltpu.async_copy(hA.at[src_v.at[c]], gbuf, sem).wait()

            @pl.when(cid == 1)
            def _():
                pltpu.async_copy(hB.at[src_v.at[c]], gbuf, sem).wait()

            pltpu.sync_copy(gbuf, acc.at[dst_v.at[c]], add=True)
            return 0

        lax.fori_loop(0, NCH_T, chunk, 0)
        plsc.subcore_barrier()

        rows = pl.ds(sid * RPT, RPT)

        @pl.when(cid == 0)
        def _():
            pltpu.sync_copy(acc.at[rows], outA.at[rows])

        @pl.when(cid == 1)
        def _():
            pltpu.sync_copy(acc.at[rows], outB.at[rows])

    return agg


@functools.lru_cache(maxsize=None)
def _make_agg16():
    mesh = plsc.VectorSubcoreMesh(core_axis_name="c", subcore_axis_name="s")

    @functools.partial(
        pl.kernel,
        out_type=(jax.ShapeDtypeStruct((NP, W16), f32),
                  jax.ShapeDtypeStruct((NP, W16), f32)),
        mesh=mesh,
        compiler_params=pltpu.CompilerParams(use_tc_tiling_on_sc=False),
        scratch_types=[
            pltpu.VMEM((NCH_W, K), jnp.int32),
            pltpu.VMEM((NCH_W, K), jnp.int32),
            pltpu.VMEM((K, W16), f32),
            pltpu.VMEM((ZROWS, W16), f32),
            pltpu.VMEM_SHARED((NP, W16), f32),
            pltpu.SemaphoreType.DMA,
        ],
    )
    def agg(h_hbm, src_hbm, dst_hbm, out0, out1,
            src_v, dst_v, gbuf, zbuf, acc, sem):
        cid = lax.axis_index("c")
        sid = lax.axis_index("s")
        wid = sid * NC + cid

        pltpu.sync_copy(src_hbm.at[wid], src_v)
        pltpu.sync_copy(dst_hbm.at[wid], dst_v)

        _zero_fill(zbuf, W16)
        _zero_shared(zbuf, acc, sid)
        plsc.subcore_barrier()

        def chunk(c, _):
            pltpu.async_copy(h_hbm.at[src_v.at[c]], gbuf, sem).wait()
            pltpu.sync_copy(gbuf, acc.at[dst_v.at[c]], add=True)
            return 0

        lax.fori_loop(0, NCH_W, chunk, 0)
        plsc.subcore_barrier()

        rows = pl.ds(sid * RPT, RPT)

        @pl.when(cid == 0)
        def _():
            pltpu.sync_copy(acc.at[rows], out0.at[rows])

        @pl.when(cid == 1)
        def _():
            pltpu.sync_copy(acc.at[rows], out1.at[rows])

    return agg


@functools.lru_cache(maxsize=None)
def _make_deg():
    mesh = plsc.VectorSubcoreMesh(core_axis_name="c", subcore_axis_name="s")

    @functools.partial(
        pl.kernel,
        out_type=(jax.ShapeDtypeStruct((NP, W16), f32),
                  jax.ShapeDtypeStruct((NP, W16), f32),
                  jax.ShapeDtypeStruct((NP, W16), f32),
                  jax.ShapeDtypeStruct((NP, W16), f32)),
        mesh=mesh,
        compiler_params=pltpu.CompilerParams(use_tc_tiling_on_sc=False),
        scratch_types=[
            pltpu.VMEM((NCH_W, K), jnp.int32),
            pltpu.VMEM((NCH_W, K), jnp.int32),
            pltpu.VMEM((K, W16), f32),
            pltpu.VMEM((ZROWS, W16), f32),
            pltpu.VMEM_SHARED((NP, W16), f32),
            pltpu.VMEM_SHARED((NP, W16), f32),
        ],
    )
    def deg(src_hbm, dst_hbm, o0, o1, i0, i1,
            src_v, dst_v, ones_v, zbuf, ohist, ihist):
        cid = lax.axis_index("c")
        sid = lax.axis_index("s")
        wid = sid * NC + cid

        pltpu.sync_copy(src_hbm.at[wid], src_v)
        pltpu.sync_copy(dst_hbm.at[wid], dst_v)

        one = jnp.ones((16,), f32)

        def orow(i, _):
            ones_v[i] = one
            return 0

        lax.fori_loop(0, K, orow, 0)
        _zero_fill(zbuf, W16)
        _zero_shared(zbuf, ohist, sid)
        _zero_shared(zbuf, ihist, sid)
        plsc.subcore_barrier()

        def chunk(c, _):
            pltpu.sync_copy(ones_v, ohist.at[src_v.at[c]], add=True)
            pltpu.sync_copy(ones_v, ihist.at[dst_v.at[c]], add=True)
            return 0

        lax.fori_loop(0, NCH_W, chunk, 0)
        plsc.subcore_barrier()

        rows = pl.ds(sid * RPT, RPT)

        @pl.when(cid == 0)
        def _():
            pltpu.sync_copy(ohist.at[rows], o0.at[rows])
            pltpu.sync_copy(ihist.at[rows], i0.at[rows])

        @pl.when(cid == 1)
        def _():
            pltpu.sync_copy(ohist.at[rows], o1.at[rows])
            pltpu.sync_copy(ihist.at[rows], i1.at[rows])

    return deg


def _norm_body(o0, o1, i0, i1, ns_ref, nd_ref):
    osum = o0[0:N, :] + o1[0:N, :]
    isum = i0[0:N, :] + i1[0:N, :]
    ns_ref[...] = lax.rsqrt(jnp.clip(osum[:, 0:1], 1.0, None))
    nd_ref[...] = lax.rsqrt(jnp.clip(isum[:, 0:1], 1.0, None))


_norm = pl.pallas_call(
    _norm_body,
    out_shape=(jax.ShapeDtypeStruct((N, 1), f32),
               jax.ShapeDtypeStruct((N, 1), f32)),
)

_half_out = (pl.BlockSpec((BLK, DH), lambda i: (i, 0)),
             pl.BlockSpec((BLK, DH), lambda i: (i, 0)))
_half_oshape = (jax.ShapeDtypeStruct((N, DH), f32),
                jax.ShapeDtypeStruct((N, DH), f32))


def _split(h, hA_ref, hB_ref):
    hA_ref[...] = h[:, 0:DH]
    hB_ref[...] = h[:, DH:D]


def _preA_body(x_ref, ns_ref, w_ref, hA_ref, hB_ref):
    _split(jnp.dot(x_ref[...] * ns_ref[...], w_ref[...],
                   preferred_element_type=f32), hA_ref, hB_ref)


_preA = pl.pallas_call(
    _preA_body,
    grid=(NBLK,),
    in_specs=[pl.BlockSpec((BLK, D), lambda i: (i, 0)),
              pl.BlockSpec((BLK, 1), lambda i: (i, 0)),
              pl.BlockSpec((D, D), lambda i: (0, 0))],
    out_specs=_half_out,
    out_shape=_half_oshape,
)


def _preB_body(pa, pb, nd, b, ns, w, hA_ref, hB_ref, hg_ref):
    i = pl.program_id(0)
    p = jnp.concatenate([pa[...], pb[...]], axis=1)
    out = jnp.maximum(p * nd[...] + b[...], 0.0)
    _split(jnp.dot(out * ns[...], w[...], preferred_element_type=f32),
           hA_ref, hB_ref)
    bs = jnp.sum(out, axis=0, keepdims=True)
    bm = jnp.max(out, axis=0, keepdims=True)

    @pl.when(i == 0)
    def _():
        hg_ref[0:1, :] = bs
        hg_ref[1:2, :] = bm

    @pl.when(i != 0)
    def _():
        hg_ref[0:1, :] = hg_ref[0:1, :] + bs
        hg_ref[1:2, :] = jnp.maximum(hg_ref[1:2, :], bm)

    @pl.when(i == NBLK - 1)
    def _():
        hg_ref[0:1, :] = hg_ref[0:1, :] * (1.0 / N)


_preB_specs = [pl.BlockSpec((BLK, DH), lambda i: (i, 0)),
               pl.BlockSpec((BLK, DH), lambda i: (i, 0)),
               pl.BlockSpec((BLK, 1), lambda i: (i, 0)),
               pl.BlockSpec((1, D), lambda i: (0, 0)),
               pl.BlockSpec((BLK, 1), lambda i: (i, 0)),
               pl.BlockSpec((D, D), lambda i: (0, 0))]

_preB = pl.pallas_call(
    _preB_body,
    grid=(NBLK,),
    in_specs=_preB_specs,
    out_specs=_half_out + (pl.BlockSpec((2, D), lambda i: (0, 0)),),
    out_shape=_half_oshape + (jax.ShapeDtypeStruct((2, D), f32),),
)


def _preC_body(pa, pb, nd, b, ns, w, out3_ref, hA_ref, hB_ref):
    p = jnp.concatenate([pa[...], pb[...]], axis=1)
    out = jnp.maximum(p * nd[...] + b[...], 0.0)
    out3_ref[...] = out
    _split(jnp.dot(out * ns[...], w[...], preferred_element_type=f32),
           hA_ref, hB_ref)


_preC = pl.pallas_call(
    _preC_body,
    grid=(NBLK,),
    in_specs=_preB_specs,
    out_specs=(pl.BlockSpec((BLK, D), lambda i: (i, 0)),) + _half_out,
    out_shape=(jax.ShapeDtypeStruct((N, D), f32),) + _half_oshape,
)


def _preD_body(pa, pb, nd, b, ns, wpad, h_ref):
    p = jnp.concatenate([pa[...], pb[...]], axis=1)
    p = p * nd[...] + b[...]
    h_ref[...] = jnp.dot(p * ns[...], wpad[...], preferred_element_type=f32)


_preD = pl.pallas_call(
    _preD_body,
    grid=(NBLK,),
    in_specs=_preB_specs[:5] + [pl.BlockSpec((D, W16), lambda i: (0, 0))],
    out_specs=pl.BlockSpec((BLK, W16), lambda i: (i, 0)),
    out_shape=jax.ShapeDtypeStruct((N, W16), f32),
)


def _final_body(pp0, pp1, nd, bdiff, gdiff, out3, hg1, hg2,
                wn0, bn0, wn1, bn1, wn2, bn2,
                wm0a, wm0b, bm0, wm1, bm1, wm2, bm2,
                pred_ref, predc_ref, hg3_ref, np_ref, acc):
    i = pl.program_id(0)
    psum = pp0[...] + pp1[...]
    dlog = (psum[:, 1:2] - psum[:, 0:1]) * nd[...] + bdiff[...]
    keep = jnp.where(dlog + gdiff[...] > 0.0, _KEEP_MASKED, 1.0)
    o3 = out3[...]
    o3m = o3 * keep

    t = jnp.maximum(jnp.dot(o3, wn0[...], preferred_element_type=f32)
                    + bn0[...], 0.0)
    t = jnp.maximum(jnp.dot(t, wn1[...], preferred_element_type=f32)
                    + bn1[...], 0.0)
    np_ref[...] = jnp.dot(t, wn2[...], preferred_element_type=f32) + bn2[...]

    bs = jnp.sum(o3m, axis=0, keepdims=True)
    bm = jnp.max(o3m, axis=0, keepdims=True)

    @pl.when(i == 0)
    def _():
        acc[0:1, :] = bs
        acc[1:2, :] = bm

    @pl.when(i != 0)
    def _():
        acc[0:1, :] = acc[0:1, :] + bs
        acc[1:2, :] = jnp.maximum(acc[1:2, :], bm)

    @pl.when(i == NBLK - 1)
    def _():
        mean3 = acc[0:1, :] * (1.0 / N)
        max3 = acc[1:2, :]
        hg3_ref[0:1, 0:D] = mean3
        hg3_ref[0:1, D:2 * D] = max3

        def mlp(hm, hx):
            h = jnp.maximum(
                jnp.dot(hm, wm0a[...], preferred_element_type=f32)
                + jnp.dot(hx, wm0b[...], preferred_element_type=f32)
                + bm0[...], 0.0)
            h = jnp.maximum(jnp.dot(h, wm1[...], preferred_element_type=f32)
                            + bm1[...], 0.0)
            return jnp.dot(h, wm2[...], preferred_element_type=f32) + bm2[...]

        pred_ref[...] = mlp(hg1[0:1, :] + hg2[0:1, :] + mean3,
                            hg1[1:2, :] + hg2[1:2, :] + max3)
        predc_ref[...] = mlp(mean3, max3)


def _const_spec(shape):
    return pl.BlockSpec(shape, lambda i: tuple(0 for _ in shape))


_final = pl.pallas_call(
    _final_body,
    grid=(NBLK,),
    in_specs=[pl.BlockSpec((BLK, W16), lambda i: (i, 0)),
              pl.BlockSpec((BLK, W16), lambda i: (i, 0)),
              pl.BlockSpec((BLK, 1), lambda i: (i, 0)),
              _const_spec((1, 1)),
              pl.BlockSpec((BLK, 1), lambda i: (i, 0)),
              pl.BlockSpec((BLK, D), lambda i: (i, 0)),
              _const_spec((2, D)),
              _const_spec((2, D)),
              _const_spec((D, D // 2)),
              _const_spec((1, D // 2)),
              _const_spec((D // 2, D // 4)),
              _const_spec((1, D // 4)),
              _const_spec((D // 4, 2)),
              _const_spec((1, 2)),
              _const_spec((D, D)),
              _const_spec((D, D)),
              _const_spec((1, D)),
              _const_spec((D, D // 2)),
              _const_spec((1, D // 2)),
              _const_spec((D // 2, 2)),
              _const_spec((1, 2))],
    out_specs=(_const_spec((1, 2)),
               _const_spec((1, 2)),
               _const_spec((1, 2 * D)),
               pl.BlockSpec((BLK, 2), lambda i: (i, 0))),
    out_shape=(jax.ShapeDtypeStruct((1, 2), f32),
               jax.ShapeDtypeStruct((1, 2), f32),
               jax.ShapeDtypeStruct((1, 2 * D), f32),
               jax.ShapeDtypeStruct((N, 2), f32)),
    scratch_shapes=[pltpu.VMEM((2, D), f32)],
)


def kernel(feature, edge_index, label, W1, b1, W2, b2, W3, b3, Wp1, bp1,
           Wp2, bp2, Wm0, bm0, Wm1, bm1, Wm2, bm2, Wn0, bn0, Wn1, bn1,
           Wn2, bn2):
    src = edge_index[0].astype(jnp.int32)
    dst = edge_index[1].astype(jnp.int32)
    src_t = src.reshape(NS, NCH_T, K)
    dst_t = dst.reshape(NS, NCH_T, K)
    src_w = src.reshape(NW, NCH_W, K)
    dst_w = dst.reshape(NW, NCH_W, K)

    _aggD = _make_agg_split()
    _agg16 = _make_agg16()

    o0, o1, i0, i1 = _make_deg()(src_w, dst_w)
    ns, nd = _norm(o0, o1, i0, i1)

    h1a, h1b = _preA(feature, ns, W1)
    p1a, p1b = _aggD(h1a, h1b, src_t, dst_t)
    h2a, h2b, hg1 = _preB(p1a, p1b, nd, b1.reshape(1, D), ns, W2)
    p2a, p2b = _aggD(h2a, h2b, src_t, dst_t)
    h3a, h3b, hg2 = _preB(p2a, p2b, nd, b2.reshape(1, D), ns, W3)
    p3a, p3b = _aggD(h3a, h3b, src_t, dst_t)
    out3, hp1a, hp1b = _preC(p3a, p3b, nd, b3.reshape(1, D), ns, Wp1)
    pp1a, pp1b = _aggD(hp1a, hp1b, src_t, dst_t)
    wp2pad = jnp.pad(Wp2, ((0, 0), (0, W16 - 2)))
    hp2 = _preD(pp1a, pp1b, nd, bp1.reshape(1, D), ns, wp2pad)
    pp2a, pp2b = _agg16(hp2, src_w, dst_w)

    u = jax.random.uniform(jax.random.key(42), (N, 2), f32, 1e-10, 1.0)
    g = -jnp.log(-jnp.log(u))
    gdiff = (g[:, 1] - g[:, 0]).reshape(N, 1)
    bdiff = (bp2[1] - bp2[0]).reshape(1, 1)

    pred, pred_com, hg3, node_pred = _final(
        pp2a, pp2b, nd, bdiff, gdiff, out3, hg1, hg2,
        Wn0, bn0.reshape(1, -1), Wn1, bn1.reshape(1, -1),
        Wn2, bn2.reshape(1, -1),
        Wm0[:D], Wm0[D:], bm0.reshape(1, D),
        Wm1, bm1.reshape(1, -1), Wm2, bm2.reshape(1, -1))
    return (pred, pred_com, hg3, node_pred)
